# baseline (speedup 1.0000x reference)
import functools

import jax
import jax.numpy as jnp
from jax.experimental import pallas as pl
from jax.experimental.pallas import tpu as pltpu


def _se_body(x_ref, w1t_ref, w2_ref, o_ref, *, inv_hw):
    bt = x_ref.shape[0]
    x = x_ref[...]
    means = jnp.sum(x, axis=-1, dtype=jnp.float32) * inv_hw
    hid = jnp.dot(means, w1t_ref[...], preferred_element_type=jnp.float32)
    hid = jnp.maximum(hid, 0.0)
    gate = jnp.dot(w2_ref[...], hid.T, preferred_element_type=jnp.float32)
    gate = jnp.clip(gate + 3.0, 0.0, 6.0) * (1.0 / 6.0)
    gate = gate.astype(x.dtype)
    for b in range(bt):
        o_ref[b] = x[b] * gate[:, b : b + 1]


def _pick_bt(B, C, HW, itemsize):
    slab = C * HW * itemsize
    budget = 11 * 1024 * 1024
    bt = 1
    while (
        bt * 2 <= 16
        and B % (bt * 2) == 0
        and B // (bt * 2) >= 4
        and 4 * (bt * 2) * slab <= budget
    ):
        bt *= 2
    return bt


def kernel(x_nchw, w1, w2):
    B, C, H, W = x_nchw.shape
    HW = H * W
    x = x_nchw.reshape(B, C, HW)
    w1t = w1.T

    bt = _pick_bt(B, C, HW, x.dtype.itemsize)
    grid = (B // bt,)

    out = pl.pallas_call(
        functools.partial(_se_body, inv_hw=1.0 / HW),
        out_shape=jax.ShapeDtypeStruct((B, C, HW), x.dtype),
        grid=grid,
        in_specs=[
            pl.BlockSpec((bt, C, HW), lambda b: (b, 0, 0)),
            pl.BlockSpec(w1t.shape, lambda b: (0, 0)),
            pl.BlockSpec(w2.shape, lambda b: (0, 0)),
        ],
        out_specs=pl.BlockSpec((bt, C, HW), lambda b: (b, 0, 0)),
        compiler_params=pltpu.CompilerParams(
            dimension_semantics=("parallel",),
            vmem_limit_bytes=56 * 1024 * 1024,
        ),
    )(x, w1t, w2)

    return out.reshape(B, C, H, W)


# speedup vs baseline: 1.0880x; 1.0880x over previous
import functools

import jax
import jax.numpy as jnp
from jax import lax
from jax.experimental import pallas as pl
from jax.experimental.pallas import tpu as pltpu

_CONTRACT_LAST = (((1,), (1,)), ((), ()))


def _se_body(x_ref, w1_ref, w2_ref, o_ref, *, inv_hw):
    bt = x_ref.shape[0]
    means = jnp.sum(x_ref[...], axis=-1, dtype=jnp.float32) * inv_hw
    hid = lax.dot_general(
        means, w1_ref[...], _CONTRACT_LAST, preferred_element_type=jnp.float32
    )
    hid = jnp.maximum(hid, 0.0)
    gate = lax.dot_general(
        w2_ref[...], hid, _CONTRACT_LAST, preferred_element_type=jnp.float32
    )
    gate = jnp.clip(gate + 3.0, 0.0, 6.0) * (1.0 / 6.0)
    gate = gate.astype(o_ref.dtype)
    for b in range(bt):
        o_ref[b] = x_ref[b] * gate[:, b : b + 1]


def _pick_bt(B, C, HW, itemsize):
    slab = C * HW * itemsize
    budget = 44 * 1024 * 1024
    bt = 1
    while (
        bt * 2 <= 16
        and B % (bt * 2) == 0
        and B // (bt * 2) >= 4
        and 4 * (bt * 2) * slab <= budget
    ):
        bt *= 2
    return bt


def kernel(x_nchw, w1, w2):
    B, C, H, W = x_nchw.shape
    HW = H * W
    x = x_nchw.reshape(B, C, HW)

    bt = _pick_bt(B, C, HW, x.dtype.itemsize)
    grid = (B // bt,)

    out = pl.pallas_call(
        functools.partial(_se_body, inv_hw=1.0 / HW),
        out_shape=jax.ShapeDtypeStruct((B, C, HW), x.dtype),
        grid=grid,
        in_specs=[
            pl.BlockSpec((bt, C, HW), lambda b: (b, 0, 0)),
            pl.BlockSpec(w1.shape, lambda b: (0, 0)),
            pl.BlockSpec(w2.shape, lambda b: (0, 0)),
        ],
        out_specs=pl.BlockSpec((bt, C, HW), lambda b: (b, 0, 0)),
        compiler_params=pltpu.CompilerParams(
            dimension_semantics=("parallel",),
            vmem_limit_bytes=56 * 1024 * 1024,
        ),
    )(x, w1, w2)

    return out.reshape(B, C, H, W)
